# baseline (speedup 1.0000x reference)
import functools

import jax
import jax.numpy as jnp
from jax import lax
from jax.experimental import pallas as pl
from jax.experimental.pallas import tpu as pltpu
from jax.experimental.pallas import tpu_sc as plsc

NR = 1000000
D = 32
NI = 524288
VR = NR // 4

NS = 16
WA = 512
NFB = NR // WA
REM_A = NFB % NS
NBA = NFB // NS
C2 = 128
NCH = NI // NS // C2

_mesh = plsc.VectorSubcoreMesh(core_axis_name="c", subcore_axis_name="s")


@functools.partial(
    pl.kernel,
    mesh=_mesh,
    out_type=(
        jax.ShapeDtypeStruct((D, 2 * NI), jnp.float32),
        jax.ShapeDtypeStruct((VR, 128), jnp.float32),
        jax.ShapeDtypeStruct((VR, 128), jnp.float32),
    ),
    scratch_types=[
        pltpu.VMEM((D, WA), jnp.float32),
        pltpu.VMEM((D, WA), jnp.float32),
        pltpu.VMEM((WA // 4, 128), jnp.float32),
        pltpu.VMEM((WA // 4, 128), jnp.float32),
        pltpu.VMEM((C2,), jnp.int32),
        pltpu.VMEM((C2,), jnp.int32),
        pltpu.VMEM((C2,), jnp.int32),
        pltpu.VMEM((C2,), jnp.int32),
        pltpu.VMEM((C2,), jnp.int32),
        pltpu.VMEM((C2,), jnp.int32),
        pltpu.VMEM((C2, 128), jnp.float32),
        pltpu.VMEM((C2, 128), jnp.float32),
        pltpu.VMEM((D, C2), jnp.float32),
        pltpu.VMEM((D, C2), jnp.float32),
        pltpu.SemaphoreType.DMA,
        pltpu.SemaphoreType.DMA,
        pltpu.SemaphoreType.DMA,
        pltpu.SemaphoreType.DMA,
        pltpu.SemaphoreType.DMA,
        pltpu.SemaphoreType.DMA,
        pltpu.SemaphoreType.DMA,
        pltpu.SemaphoreType.DMA,
        pltpu.SemaphoreType.DMA,
        pltpu.SemaphoreType.DMA,
    ],
    compiler_params=pltpu.CompilerParams(needs_layout_passes=False),
)
def _gather_concat(ft0, ft1, i0, i1, tl0, tl1, ot, s0, s1,
                   bufA0, bufA1, bufT0, bufT1,
                   idx0b, idx1b, vi0b, vi1b, lo0b, lo1b,
                   grows0, grows1, otb0, otb1,
                   rs0, rs1, ws0, ws1, gs0, gs1, ow0, ow1, is0, is1):
    core = lax.axis_index("c")
    t = lax.axis_index("s")
    iota = lax.iota(jnp.int32, 16)

    bufA = (bufA0, bufA1)
    bufT = (bufT0, bufT1)
    idxb = (idx0b, idx1b)
    vib = (vi0b, vi1b)
    lob = (lo0b, lo1b)
    growsb = (grows0, grows1)
    otb = (otb0, otb1)
    rs = (rs0, rs1)
    ws = (ws0, ws1)
    gs = (gs0, gs1)
    ow = (ow0, ow1)
    isx = (is0, is1)

    iotad4 = jax.lax.shift_right_logical(iota, 2)
    iotam4 = jax.lax.rem(iota, 4)
    rowvs = [iotad4 + g2 * 4 for g2 in range(8)]

    def transpose_block(bA, bT):
        def v_body(v4, carry):
            for dv in range(4):
                v = v4 * 4 + dv
                colv = iotam4 + 4 * v
                for g2 in range(8):
                    w = plsc.load_gather(bA, [rowvs[g2], colv])
                    bT[v, pl.ds(g2 * 16, 16)] = w
            return carry
        lax.fori_loop(0, WA // 16, v_body, 0)

    def stage_a(ft, s):
        nbt = jnp.where(t < REM_A, NBA + 1, NBA)

        def rd_start(m, b):
            r0 = pl.multiple_of((t + NS * m) * WA, WA)
            pltpu.async_copy(ft.at[:, pl.ds(r0, WA)], bufA[b], rs[b])

        def rd_wait(b):
            pltpu.make_async_copy(ft.at[:, pl.ds(0, WA)], bufA[b],
                                  rs[b]).wait()

        def wr_start(m, b):
            v0 = pl.multiple_of((t + NS * m) * (WA // 4), WA // 4)
            pltpu.async_copy(bufT[b], s.at[pl.ds(v0, WA // 4)], ws[b])

        def wr_wait(b):
            pltpu.make_async_copy(bufT[b], s.at[pl.ds(0, WA // 4)],
                                  ws[b]).wait()

        rd_start(0, 0)
        rd_start(1, 1)

        def pair_body(mm, carry):
            for b in (0, 1):
                m = 2 * mm + b

                @pl.when(m < nbt)
                def _():
                    rd_wait(b)

                    @pl.when(m >= 2)
                    def _():
                        wr_wait(b)

                    transpose_block(bufA[b], bufT[b])
                    wr_start(m, b)

                    @pl.when(m + 2 < nbt)
                    def _():
                        rd_start(m + 2, b)
            return carry
        lax.fori_loop(0, (NBA + 2) // 2, pair_body, 0)
        wr_wait(0)
        wr_wait(1)

    def tail_copy(tl, s):
        pltpu.sync_copy(tl, bufT0.at[pl.ds(0, 16)])
        pltpu.sync_copy(bufT0.at[pl.ds(0, 16)], s.at[pl.ds(NFB * (WA // 4), 16)])

    def stage_b(ih, s, ocol0):
        jbase = t * (NI // NS)

        def i_start(k, b):
            ioff = pl.multiple_of(jbase + k * C2, C2)
            pltpu.async_copy(ih.at[pl.ds(ioff, C2)], idxb[b], isx[b])

        def i_wait(b):
            pltpu.make_async_copy(ih.at[pl.ds(0, C2)], idxb[b],
                                  isx[b]).wait()

        def pre_and_gather(b):
            def pre_body(j, c2):
                v = idxb[b][pl.ds(j * 16, 16)]
                vib[b][pl.ds(j * 16, 16)] = jax.lax.shift_right_logical(v, 2)
                lob[b][pl.ds(j * 16, 16)] = jax.lax.rem(v, 4)
                return c2
            lax.fori_loop(0, C2 // 16, pre_body, 0)
            pltpu.async_copy(s.at[vib[b]], growsb[b], gs[b])

        def g_wait(b):
            pltpu.make_async_copy(s.at[vib[b]], growsb[b], gs[b]).wait()

        def extract(b):
            def ex_body(g, c2):
                j0 = g * 16
                jvec = iota + j0
                lo16 = lob[b][pl.ds(j0, 16)]
                for c in range(D):
                    w = plsc.load_gather(growsb[b], [jvec, lo16 + 4 * c])
                    otb[b][c, pl.ds(j0, 16)] = w
                return c2
            lax.fori_loop(0, C2 // 16, ex_body, 0)

        def w_start(k, b):
            ocol = pl.multiple_of(ocol0 + jbase + k * C2, C2)
            pltpu.async_copy(otb[b], ot.at[:, pl.ds(ocol, C2)], ow[b])

        def w_wait(b):
            pltpu.make_async_copy(otb[b], ot.at[:, pl.ds(ocol0, C2)],
                                  ow[b]).wait()

        i_start(0, 0)
        i_wait(0)
        pre_and_gather(0)
        i_start(1, 1)

        def pair_body(kk, carry):
            for b in (0, 1):
                k = 2 * kk + b
                nb = 1 - b

                @pl.when(k + 1 < NCH)
                def _():
                    i_wait(nb)
                    pre_and_gather(nb)

                @pl.when(k + 2 < NCH)
                def _():
                    i_start(k + 2, b)

                g_wait(b)

                @pl.when(k >= 2)
                def _():
                    w_wait(b)

                extract(b)
                w_start(k, b)
            return carry
        lax.fori_loop(0, NCH // 2, pair_body, 0)
        w_wait(0)
        w_wait(1)

    @pl.when(core == 0)
    def _c0a():
        stage_a(ft0, s0)

        @pl.when(t == NS - 1)
        def _():
            tail_copy(tl0, s0)

    @pl.when(core == 1)
    def _c1a():
        stage_a(ft1, s1)

        @pl.when(t == NS - 1)
        def _():
            tail_copy(tl1, s1)

    plsc.subcore_barrier()

    @pl.when(core == 0)
    def _c0b():
        stage_b(i0, s0, 0)

    @pl.when(core == 1)
    def _c1b():
        stage_b(i1, s1, NI)


def kernel(feat_0, feat_1, index_map_0, index_map_1):
    def mk_tail(feat):
        return feat[NFB * WA:].reshape(16, 4, 32).transpose(0, 2, 1).reshape(16, 128)
    tl0 = mk_tail(feat_0)
    tl1 = mk_tail(feat_1)
    ot, _, _ = _gather_concat(feat_0.T, feat_1.T,
                              index_map_0.astype(jnp.int32),
                              index_map_1.astype(jnp.int32),
                              tl0, tl1)
    return ot.T


# speedup vs baseline: 1.0028x; 1.0028x over previous
import functools

import jax
import jax.numpy as jnp
from jax import lax
from jax.experimental import pallas as pl
from jax.experimental.pallas import tpu as pltpu
from jax.experimental.pallas import tpu_sc as plsc

NR = 1000000
D = 32
NI = 524288
VR = NR // 4

NS = 16
WA = 256
NFB = NR // WA
REM_A = NFB % NS
NBA = NFB // NS
C2 = 256
NCH = NI // NS // C2

_mesh = plsc.VectorSubcoreMesh(core_axis_name="c", subcore_axis_name="s")


@functools.partial(
    pl.kernel,
    mesh=_mesh,
    out_type=(
        jax.ShapeDtypeStruct((D, 2 * NI), jnp.float32),
        jax.ShapeDtypeStruct((VR, 128), jnp.float32),
        jax.ShapeDtypeStruct((VR, 128), jnp.float32),
    ),
    scratch_types=[
        pltpu.VMEM((D, WA), jnp.float32),
        pltpu.VMEM((D, WA), jnp.float32),
        pltpu.VMEM((WA // 4, 128), jnp.float32),
        pltpu.VMEM((WA // 4, 128), jnp.float32),
        pltpu.VMEM((C2,), jnp.int32),
        pltpu.VMEM((C2,), jnp.int32),
        pltpu.VMEM((C2,), jnp.int32),
        pltpu.VMEM((C2,), jnp.int32),
        pltpu.VMEM((C2,), jnp.int32),
        pltpu.VMEM((C2,), jnp.int32),
        pltpu.VMEM((C2, 128), jnp.float32),
        pltpu.VMEM((C2, 128), jnp.float32),
        pltpu.VMEM((D, C2), jnp.float32),
        pltpu.VMEM((D, C2), jnp.float32),
        pltpu.SemaphoreType.DMA,
        pltpu.SemaphoreType.DMA,
        pltpu.SemaphoreType.DMA,
        pltpu.SemaphoreType.DMA,
        pltpu.SemaphoreType.DMA,
        pltpu.SemaphoreType.DMA,
        pltpu.SemaphoreType.DMA,
        pltpu.SemaphoreType.DMA,
        pltpu.SemaphoreType.DMA,
        pltpu.SemaphoreType.DMA,
    ],
    compiler_params=pltpu.CompilerParams(needs_layout_passes=False),
)
def _gather_concat(ft0, ft1, i0, i1, tl0, tl1, ot, s0, s1,
                   bufA0, bufA1, bufT0, bufT1,
                   idx0b, idx1b, vi0b, vi1b, lo0b, lo1b,
                   grows0, grows1, otb0, otb1,
                   rs0, rs1, ws0, ws1, gs0, gs1, ow0, ow1, is0, is1):
    core = lax.axis_index("c")
    t = lax.axis_index("s")
    iota = lax.iota(jnp.int32, 16)

    bufA = (bufA0, bufA1)
    bufT = (bufT0, bufT1)
    idxb = (idx0b, idx1b)
    vib = (vi0b, vi1b)
    lob = (lo0b, lo1b)
    growsb = (grows0, grows1)
    otb = (otb0, otb1)
    rs = (rs0, rs1)
    ws = (ws0, ws1)
    gs = (gs0, gs1)
    ow = (ow0, ow1)
    isx = (is0, is1)

    iotad4 = jax.lax.shift_right_logical(iota, 2)
    iotam4 = jax.lax.rem(iota, 4)
    rowvs = [iotad4 + g2 * 4 for g2 in range(8)]

    def transpose_block(bA, bT):
        def v_body(v4, carry):
            for dv in range(4):
                v = v4 * 4 + dv
                colv = iotam4 + 4 * v
                for g2 in range(8):
                    w = plsc.load_gather(bA, [rowvs[g2], colv])
                    bT[v, pl.ds(g2 * 16, 16)] = w
            return carry
        lax.fori_loop(0, WA // 16, v_body, 0)

    def stage_a(ft, s):
        nbt = jnp.where(t < REM_A, NBA + 1, NBA)

        def rd_start(m, b):
            r0 = pl.multiple_of((t + NS * m) * WA, WA)
            pltpu.async_copy(ft.at[:, pl.ds(r0, WA)], bufA[b], rs[b])

        def rd_wait(b):
            pltpu.make_async_copy(ft.at[:, pl.ds(0, WA)], bufA[b],
                                  rs[b]).wait()

        def wr_start(m, b):
            v0 = pl.multiple_of((t + NS * m) * (WA // 4), WA // 4)
            pltpu.async_copy(bufT[b], s.at[pl.ds(v0, WA // 4)], ws[b])

        def wr_wait(b):
            pltpu.make_async_copy(bufT[b], s.at[pl.ds(0, WA // 4)],
                                  ws[b]).wait()

        rd_start(0, 0)
        rd_start(1, 1)

        def pair_body(mm, carry):
            for b in (0, 1):
                m = 2 * mm + b

                @pl.when(m < nbt)
                def _():
                    rd_wait(b)

                    @pl.when(m >= 2)
                    def _():
                        wr_wait(b)

                    transpose_block(bufA[b], bufT[b])
                    wr_start(m, b)

                    @pl.when(m + 2 < nbt)
                    def _():
                        rd_start(m + 2, b)
            return carry
        lax.fori_loop(0, (NBA + 2) // 2, pair_body, 0)
        wr_wait(0)
        wr_wait(1)

    def tail_copy(tl, s):
        pltpu.sync_copy(tl, bufT0.at[pl.ds(0, 16)])
        pltpu.sync_copy(bufT0.at[pl.ds(0, 16)], s.at[pl.ds(NFB * (WA // 4), 16)])

    def stage_b(ih, s, ocol0):
        jbase = t * (NI // NS)

        def i_start(k, b):
            ioff = pl.multiple_of(jbase + k * C2, C2)
            pltpu.async_copy(ih.at[pl.ds(ioff, C2)], idxb[b], isx[b])

        def i_wait(b):
            pltpu.make_async_copy(ih.at[pl.ds(0, C2)], idxb[b],
                                  isx[b]).wait()

        def pre_and_gather(b):
            def pre_body(j, c2):
                v = idxb[b][pl.ds(j * 16, 16)]
                vib[b][pl.ds(j * 16, 16)] = jax.lax.shift_right_logical(v, 2)
                lob[b][pl.ds(j * 16, 16)] = jax.lax.rem(v, 4)
                return c2
            lax.fori_loop(0, C2 // 16, pre_body, 0)
            pltpu.async_copy(s.at[vib[b]], growsb[b], gs[b])

        def g_wait(b):
            pltpu.make_async_copy(s.at[vib[b]], growsb[b], gs[b]).wait()

        def extract(b):
            def ex_body(g, c2):
                j0 = g * 16
                jvec = iota + j0
                lo16 = lob[b][pl.ds(j0, 16)]
                for c in range(D):
                    w = plsc.load_gather(growsb[b], [jvec, lo16 + 4 * c])
                    otb[b][c, pl.ds(j0, 16)] = w
                return c2
            lax.fori_loop(0, C2 // 16, ex_body, 0)

        def w_start(k, b):
            ocol = pl.multiple_of(ocol0 + jbase + k * C2, C2)
            pltpu.async_copy(otb[b], ot.at[:, pl.ds(ocol, C2)], ow[b])

        def w_wait(b):
            pltpu.make_async_copy(otb[b], ot.at[:, pl.ds(ocol0, C2)],
                                  ow[b]).wait()

        i_start(0, 0)
        i_wait(0)
        pre_and_gather(0)
        i_start(1, 1)

        def pair_body(kk, carry):
            for b in (0, 1):
                k = 2 * kk + b
                nb = 1 - b

                @pl.when(k + 1 < NCH)
                def _():
                    i_wait(nb)
                    pre_and_gather(nb)

                @pl.when(k + 2 < NCH)
                def _():
                    i_start(k + 2, b)

                g_wait(b)

                @pl.when(k >= 2)
                def _():
                    w_wait(b)

                extract(b)
                w_start(k, b)
            return carry
        lax.fori_loop(0, NCH // 2, pair_body, 0)
        w_wait(0)
        w_wait(1)

    @pl.when(core == 0)
    def _c0a():
        stage_a(ft0, s0)

        @pl.when(t == NS - 1)
        def _():
            tail_copy(tl0, s0)

    @pl.when(core == 1)
    def _c1a():
        stage_a(ft1, s1)

        @pl.when(t == NS - 1)
        def _():
            tail_copy(tl1, s1)

    plsc.subcore_barrier()

    @pl.when(core == 0)
    def _c0b():
        stage_b(i0, s0, 0)

    @pl.when(core == 1)
    def _c1b():
        stage_b(i1, s1, NI)


def kernel(feat_0, feat_1, index_map_0, index_map_1):
    def mk_tail(feat):
        return feat[NFB * WA:].reshape(16, 4, 32).transpose(0, 2, 1).reshape(16, 128)
    tl0 = mk_tail(feat_0)
    tl1 = mk_tail(feat_1)
    ot, _, _ = _gather_concat(feat_0.T, feat_1.T,
                              index_map_0.astype(jnp.int32),
                              index_map_1.astype(jnp.int32),
                              tl0, tl1)
    return ot.T
